# Initial kernel scaffold; baseline (speedup 1.0000x reference)
import jax
import jax.numpy as jnp
from jax.experimental import pallas as pl


def kernel(w_hat, neuron_to_edge_map, firing_rate_ema):
    raise NotImplementedError("write your pallas kernel here")


# speedup vs baseline: 318.4696x; 318.4696x over previous
import functools

import jax
import jax.numpy as jnp
from jax import lax
from jax.experimental import pallas as pl
from jax.experimental.pallas import tpu as pltpu
from jax.experimental.pallas import tpu_sc as plsc

_NUM_NEURONS = 100000
_NUM_EDGES = 6400000
_TARGET_RATE = 0.05
_SCALING_STRENGTH = 0.1

_LANES = 128
_PAD_NEURONS = 100352

_NW = 32
_EDGES_PER_W = _NUM_EDGES // _NW
_CHUNK = 4000
_NCHUNK = _EDGES_PER_W // _CHUNK
_VPC = _CHUNK // 16


def _scale_body(ema_ref, out_ref):
    rate = jnp.maximum(ema_ref[...], 1e-6)
    ratio = _TARGET_RATE / rate
    s = jnp.exp(_SCALING_STRENGTH * jnp.log(ratio))
    out_ref[...] = jnp.clip(s, 0.5, 2.0)


def _compute_scale(ema):
    ema2d = jnp.pad(ema, (0, _PAD_NEURONS - _NUM_NEURONS)).reshape(
        _PAD_NEURONS // _LANES, _LANES)
    out = pl.pallas_call(
        _scale_body,
        out_shape=jax.ShapeDtypeStruct(ema2d.shape, jnp.float32),
    )(ema2d)
    return out.reshape(_PAD_NEURONS)[:_NUM_NEURONS]


def _gather_body(scale_hbm, idx_hbm, w_hbm, out_hbm, table_v, idx_v, w_v):
    wid = lax.axis_index("s") * 2 + lax.axis_index("c")
    base = wid * _EDGES_PER_W

    pltpu.sync_copy(scale_hbm, table_v)

    def chunk_body(g, carry):
        off = base + g * _CHUNK
        pltpu.sync_copy(idx_hbm.at[pl.ds(off, _CHUNK)], idx_v)
        pltpu.sync_copy(w_hbm.at[pl.ds(off, _CHUNK)], w_v)

        def vec_body(i, c):
            sl = pl.ds(i * 16, 16)
            iv = idx_v[sl]
            sv = plsc.load_gather(table_v, [iv])
            w_v[sl] = w_v[sl] * sv
            return c

        lax.fori_loop(0, _VPC, vec_body, 0)
        pltpu.sync_copy(w_v, out_hbm.at[pl.ds(off, _CHUNK)])
        return carry

    lax.fori_loop(0, _NCHUNK, chunk_body, 0)


@functools.partial(
    pl.kernel,
    out_type=jax.ShapeDtypeStruct((_NUM_EDGES,), jnp.float32),
    mesh=plsc.VectorSubcoreMesh(core_axis_name="c", subcore_axis_name="s"),
    compiler_params=pltpu.CompilerParams(needs_layout_passes=False),
    scratch_types=[
        pltpu.VMEM((_NUM_NEURONS,), jnp.float32),
        pltpu.VMEM((_CHUNK,), jnp.int32),
        pltpu.VMEM((_CHUNK,), jnp.float32),
    ],
)
def _sc_gather(scale_hbm, idx_hbm, w_hbm, out_hbm, table_v, idx_v, w_v):
    _gather_body(scale_hbm, idx_hbm, w_hbm, out_hbm, table_v, idx_v, w_v)


def kernel(w_hat, neuron_to_edge_map, firing_rate_ema):
    scale = _compute_scale(firing_rate_ema)
    return _sc_gather(scale, neuron_to_edge_map, w_hat)


# speedup vs baseline: 864.1020x; 2.7133x over previous
import functools

import jax
import jax.numpy as jnp
from jax import lax
from jax.experimental import pallas as pl
from jax.experimental.pallas import tpu as pltpu
from jax.experimental.pallas import tpu_sc as plsc

_NUM_NEURONS = 100000
_NUM_EDGES = 6400000
_TARGET_RATE = 0.05
_SCALING_STRENGTH = 0.1

_LANES = 128
_PAD_NEURONS = 100352

_NW = 32
_EDGES_PER_W = _NUM_EDGES // _NW
_CHUNK = 4000
_NCHUNK = _EDGES_PER_W // _CHUNK
_VPC = _CHUNK // 16
_UNROLL = 10


def _scale_body(ema_ref, out_ref):
    rate = jnp.maximum(ema_ref[...], 1e-6)
    ratio = _TARGET_RATE / rate
    s = jnp.exp(_SCALING_STRENGTH * jnp.log(ratio))
    out_ref[...] = jnp.clip(s, 0.5, 2.0)


def _compute_scale(ema):
    ema2d = jnp.pad(ema, (0, _PAD_NEURONS - _NUM_NEURONS)).reshape(
        _PAD_NEURONS // _LANES, _LANES)
    out = pl.pallas_call(
        _scale_body,
        out_shape=jax.ShapeDtypeStruct(ema2d.shape, jnp.float32),
    )(ema2d)
    return out.reshape(_PAD_NEURONS)[:_NUM_NEURONS]


def _gather_body(scale_hbm, idx_hbm, w_hbm, out_hbm,
                 table_v, idx_v0, idx_v1, w_v0, w_v1,
                 sem_t, sem_i, sem_w, sem_o):
    wid = lax.axis_index("s") * 2 + lax.axis_index("c")
    base = wid * _EDGES_PER_W
    bufs = ((idx_v0, w_v0), (idx_v1, w_v1))

    ct = pltpu.async_copy(scale_hbm, table_v, sem_t)
    pltpu.async_copy(idx_hbm.at[pl.ds(base, _CHUNK)], idx_v0, sem_i)
    pltpu.async_copy(w_hbm.at[pl.ds(base, _CHUNK)], w_v0, sem_w)
    ct.wait()

    def pair_body(g2, carry):
        for u in range(2):
            g = g2 * 2 + u
            ib, wb = bufs[u]
            nib, nwb = bufs[1 - u]
            off = base + g * _CHUNK

            @pl.when(g + 1 < _NCHUNK)
            def _():
                @pl.when(g >= 1)
                def _():
                    pltpu.make_async_copy(
                        nwb, out_hbm.at[pl.ds(off - _CHUNK, _CHUNK)],
                        sem_o).wait()
                pltpu.async_copy(
                    idx_hbm.at[pl.ds(off + _CHUNK, _CHUNK)], nib, sem_i)
                pltpu.async_copy(
                    w_hbm.at[pl.ds(off + _CHUNK, _CHUNK)], nwb, sem_w)

            pltpu.make_async_copy(
                idx_hbm.at[pl.ds(off, _CHUNK)], ib, sem_i).wait()
            pltpu.make_async_copy(
                w_hbm.at[pl.ds(off, _CHUNK)], wb, sem_w).wait()

            @plsc.parallel_loop(0, _VPC, unroll=_UNROLL)
            def _(i):
                sl = pl.ds(i * 16, 16)
                sv = plsc.load_gather(table_v, [ib[sl]])
                wb[sl] = wb[sl] * sv

            pltpu.async_copy(wb, out_hbm.at[pl.ds(off, _CHUNK)], sem_o)
        return carry

    lax.fori_loop(0, _NCHUNK // 2, pair_body, 0)

    last = _NCHUNK - 1
    pltpu.make_async_copy(
        bufs[last % 2][1],
        out_hbm.at[pl.ds(base + last * _CHUNK, _CHUNK)], sem_o).wait()


@functools.partial(
    pl.kernel,
    out_type=jax.ShapeDtypeStruct((_NUM_EDGES,), jnp.float32),
    mesh=plsc.VectorSubcoreMesh(core_axis_name="c", subcore_axis_name="s"),
    compiler_params=pltpu.CompilerParams(needs_layout_passes=False),
    scratch_types=[
        pltpu.VMEM((_NUM_NEURONS,), jnp.float32),
        pltpu.VMEM((_CHUNK,), jnp.int32),
        pltpu.VMEM((_CHUNK,), jnp.int32),
        pltpu.VMEM((_CHUNK,), jnp.float32),
        pltpu.VMEM((_CHUNK,), jnp.float32),
        pltpu.SemaphoreType.DMA,
        pltpu.SemaphoreType.DMA,
        pltpu.SemaphoreType.DMA,
        pltpu.SemaphoreType.DMA,
    ],
)
def _sc_gather(scale_hbm, idx_hbm, w_hbm, out_hbm,
               table_v, idx_v0, idx_v1, w_v0, w_v1,
               sem_t, sem_i, sem_w, sem_o):
    _gather_body(scale_hbm, idx_hbm, w_hbm, out_hbm,
                 table_v, idx_v0, idx_v1, w_v0, w_v1,
                 sem_t, sem_i, sem_w, sem_o)


def kernel(w_hat, neuron_to_edge_map, firing_rate_ema):
    scale = _compute_scale(firing_rate_ema)
    return _sc_gather(scale, neuron_to_edge_map, w_hat)


# speedup vs baseline: 866.5330x; 1.0028x over previous
import functools

import jax
import jax.numpy as jnp
from jax import lax
from jax.experimental import pallas as pl
from jax.experimental.pallas import tpu as pltpu
from jax.experimental.pallas import tpu_sc as plsc

_NUM_NEURONS = 100000
_NUM_EDGES = 6400000
_TARGET_RATE = 0.05
_SCALING_STRENGTH = 0.1

_LANES = 128
_PAD_NEURONS = 100352

_NW = 32
_EDGES_PER_W = _NUM_EDGES // _NW
_CHUNK = 4000
_NCHUNK = _EDGES_PER_W // _CHUNK
_VPC = _CHUNK // 16
_UNROLL = 25


def _scale_body(ema_ref, out_ref):
    rate = jnp.maximum(ema_ref[...], 1e-6)
    ratio = _TARGET_RATE / rate
    s = jnp.exp(_SCALING_STRENGTH * jnp.log(ratio))
    out_ref[...] = jnp.clip(s, 0.5, 2.0)


def _compute_scale(ema):
    ema2d = jnp.pad(ema, (0, _PAD_NEURONS - _NUM_NEURONS)).reshape(
        _PAD_NEURONS // _LANES, _LANES)
    out = pl.pallas_call(
        _scale_body,
        out_shape=jax.ShapeDtypeStruct(ema2d.shape, jnp.float32),
    )(ema2d)
    return out.reshape(_PAD_NEURONS)[:_NUM_NEURONS]


def _gather_body(scale_hbm, idx_hbm, w_hbm, out_hbm,
                 table_v, idx_v0, idx_v1, w_v0, w_v1,
                 sem_t, sem_i, sem_w, sem_o):
    wid = lax.axis_index("s") * 2 + lax.axis_index("c")
    base = wid * _EDGES_PER_W
    bufs = ((idx_v0, w_v0), (idx_v1, w_v1))

    ct = pltpu.async_copy(scale_hbm, table_v, sem_t)
    pltpu.async_copy(idx_hbm.at[pl.ds(base, _CHUNK)], idx_v0, sem_i)
    pltpu.async_copy(w_hbm.at[pl.ds(base, _CHUNK)], w_v0, sem_w)
    ct.wait()

    def pair_body(g2, carry):
        for u in range(2):
            g = g2 * 2 + u
            ib, wb = bufs[u]
            nib, nwb = bufs[1 - u]
            off = base + g * _CHUNK

            @pl.when(g + 1 < _NCHUNK)
            def _():
                @pl.when(g >= 1)
                def _():
                    pltpu.make_async_copy(
                        nwb, out_hbm.at[pl.ds(off - _CHUNK, _CHUNK)],
                        sem_o).wait()
                pltpu.async_copy(
                    idx_hbm.at[pl.ds(off + _CHUNK, _CHUNK)], nib, sem_i)
                pltpu.async_copy(
                    w_hbm.at[pl.ds(off + _CHUNK, _CHUNK)], nwb, sem_w)

            pltpu.make_async_copy(
                idx_hbm.at[pl.ds(off, _CHUNK)], ib, sem_i).wait()
            pltpu.make_async_copy(
                w_hbm.at[pl.ds(off, _CHUNK)], wb, sem_w).wait()

            @plsc.parallel_loop(0, _VPC, unroll=_UNROLL)
            def _(i):
                sl = pl.ds(i * 16, 16)
                sv = plsc.load_gather(table_v, [ib[sl]])
                wb[sl] = wb[sl] * sv

            pltpu.async_copy(wb, out_hbm.at[pl.ds(off, _CHUNK)], sem_o)
        return carry

    lax.fori_loop(0, _NCHUNK // 2, pair_body, 0)

    last = _NCHUNK - 1
    pltpu.make_async_copy(
        bufs[last % 2][1],
        out_hbm.at[pl.ds(base + last * _CHUNK, _CHUNK)], sem_o).wait()


@functools.partial(
    pl.kernel,
    out_type=jax.ShapeDtypeStruct((_NUM_EDGES,), jnp.float32),
    mesh=plsc.VectorSubcoreMesh(core_axis_name="c", subcore_axis_name="s"),
    compiler_params=pltpu.CompilerParams(needs_layout_passes=False),
    scratch_types=[
        pltpu.VMEM((_NUM_NEURONS,), jnp.float32),
        pltpu.VMEM((_CHUNK,), jnp.int32),
        pltpu.VMEM((_CHUNK,), jnp.int32),
        pltpu.VMEM((_CHUNK,), jnp.float32),
        pltpu.VMEM((_CHUNK,), jnp.float32),
        pltpu.SemaphoreType.DMA,
        pltpu.SemaphoreType.DMA,
        pltpu.SemaphoreType.DMA,
        pltpu.SemaphoreType.DMA,
    ],
)
def _sc_gather(scale_hbm, idx_hbm, w_hbm, out_hbm,
               table_v, idx_v0, idx_v1, w_v0, w_v1,
               sem_t, sem_i, sem_w, sem_o):
    _gather_body(scale_hbm, idx_hbm, w_hbm, out_hbm,
                 table_v, idx_v0, idx_v1, w_v0, w_v1,
                 sem_t, sem_i, sem_w, sem_o)


def kernel(w_hat, neuron_to_edge_map, firing_rate_ema):
    scale = _compute_scale(firing_rate_ema)
    return _sc_gather(scale, neuron_to_edge_map, w_hat)
